# Initial kernel scaffold; baseline (speedup 1.0000x reference)
import jax
import jax.numpy as jnp
from jax.experimental import pallas as pl


def kernel(uid_batch_ph, mid_batch_ph, cate_batch_ph, mid_his_batch_ph, cate_his_batch_ph, mask, uid_table, mid_table, cate_table):
    raise NotImplementedError("write your pallas kernel here")


# speedup vs baseline: 2.9224x; 2.9224x over previous
import functools

import jax
import jax.numpy as jnp
from jax import lax
from jax.experimental import pallas as pl
from jax.experimental.pallas import tpu as pltpu
from jax.experimental.pallas import tpu_sc as plsc

_LANES = 16


def _build_sc_kernel(B, L, EMB, n_uid, n_mid, n_cate):
    info = plsc.get_sparse_core_info()
    NC, NS = info.num_cores, info.num_subcores
    NW = NC * NS
    EPW = B // NW
    SUB = 32
    NSUB = EPW // SUB
    LH = L // 2
    OUTW = 5 * EMB

    mesh = plsc.VectorSubcoreMesh(core_axis_name="c", subcore_axis_name="s")

    @functools.partial(
        pl.kernel,
        mesh=mesh,
        out_type=jax.ShapeDtypeStruct((B, OUTW), jnp.float32),
        compiler_params=pltpu.CompilerParams(use_tc_tiling_on_sc=False),
        scratch_types=[
            pltpu.VMEM((SUB, 2, LH), jnp.int32),
            pltpu.VMEM((SUB, 2, LH), jnp.int32),
            pltpu.VMEM((SUB, L), jnp.float32),
            pltpu.VMEM((L, EMB), jnp.float32),
            pltpu.VMEM((L, EMB), jnp.float32),
            pltpu.VMEM((EPW, OUTW), jnp.float32),
            pltpu.VMEM((EPW,), jnp.int32),
            pltpu.VMEM((EPW,), jnp.int32),
            pltpu.VMEM((EPW,), jnp.int32),
            pltpu.VMEM((EPW, EMB), jnp.float32),
            pltpu.VMEM((EPW, EMB), jnp.float32),
            pltpu.VMEM((EPW, EMB), jnp.float32),
            pltpu.SemaphoreType.DMA,
            pltpu.SemaphoreType.DMA,
        ],
    )
    def sc_kernel(uid_i_hbm, mid_i_hbm, cate_i_hbm, mh_hbm, ch_hbm, mk_hbm,
                  uid_t_hbm, mid_t_hbm, cate_t_hbm, out_hbm,
                  mh_v, ch_v, mk_v, r_mid, r_cate, out_v,
                  uidx_v, midx_v, cidx_v, urows, mrows, crows,
                  sem_his, sem_one):
        wid = lax.axis_index("s") * NC + lax.axis_index("c")
        base = wid * EPW

        pltpu.sync_copy(uid_i_hbm.at[pl.ds(base, EPW)], uidx_v)
        pltpu.sync_copy(mid_i_hbm.at[pl.ds(base, EPW)], midx_v)
        pltpu.sync_copy(cate_i_hbm.at[pl.ds(base, EPW)], cidx_v)
        cu = pltpu.async_copy(uid_t_hbm.at[uidx_v], urows, sem_one)
        cm = pltpu.async_copy(mid_t_hbm.at[midx_v], mrows, sem_one)
        cc = pltpu.async_copy(cate_t_hbm.at[cidx_v], crows, sem_one)
        cu.wait()
        cm.wait()
        cc.wait()

        for sc in range(NSUB):
            cb = base + sc * SUB
            pltpu.sync_copy(mh_hbm.at[pl.ds(cb, SUB)], mh_v)
            pltpu.sync_copy(ch_hbm.at[pl.ds(cb, SUB)], ch_v)
            pltpu.sync_copy(mk_hbm.at[pl.ds(cb, SUB)], mk_v)

            def el_body(el, _, sc=sc):
                g = sc * SUB + el
                c1 = pltpu.async_copy(mid_t_hbm.at[mh_v.at[el, 0]],
                                      r_mid.at[pl.ds(0, LH)], sem_his)
                c2 = pltpu.async_copy(mid_t_hbm.at[mh_v.at[el, 1]],
                                      r_mid.at[pl.ds(LH, LH)], sem_his)
                c3 = pltpu.async_copy(cate_t_hbm.at[ch_v.at[el, 0]],
                                      r_cate.at[pl.ds(0, LH)], sem_his)
                c4 = pltpu.async_copy(cate_t_hbm.at[ch_v.at[el, 1]],
                                      r_cate.at[pl.ds(LH, LH)], sem_his)
                c1.wait()
                c2.wait()
                c3.wait()
                c4.wait()

                zero = jnp.zeros((_LANES,), jnp.float32)

                def red_body(i, carry):
                    a0, a1, b0, b1 = carry
                    lb = i * _LANES
                    mvec = mk_v[el, pl.ds(lb, _LANES)]
                    for j in range(_LANES):
                        l = lb + j
                        m = mvec[j]
                        a0 = a0 + m * r_mid[l, pl.ds(0, _LANES)]
                        a1 = a1 + m * r_mid[l, pl.ds(_LANES, _LANES)]
                        b0 = b0 + m * r_cate[l, pl.ds(0, _LANES)]
                        b1 = b1 + m * r_cate[l, pl.ds(_LANES, _LANES)]
                    return a0, a1, b0, b1

                nfull = L // _LANES
                a0, a1, b0, b1 = lax.fori_loop(
                    0, nfull, red_body, (zero, zero, zero, zero))
                ntail = L - nfull * _LANES
                if ntail:
                    mvec = mk_v[el, pl.ds(L - _LANES, _LANES)]
                    for j in range(ntail):
                        l = nfull * _LANES + j
                        m = mvec[_LANES - ntail + j]
                        a0 = a0 + m * r_mid[l, pl.ds(0, _LANES)]
                        a1 = a1 + m * r_mid[l, pl.ds(_LANES, _LANES)]
                        b0 = b0 + m * r_cate[l, pl.ds(0, _LANES)]
                        b1 = b1 + m * r_cate[l, pl.ds(_LANES, _LANES)]

                out_v[g, pl.ds(0, _LANES)] = urows[g, pl.ds(0, _LANES)]
                out_v[g, pl.ds(16, _LANES)] = urows[g, pl.ds(_LANES, _LANES)]
                out_v[g, pl.ds(32, _LANES)] = mrows[g, pl.ds(0, _LANES)]
                out_v[g, pl.ds(48, _LANES)] = mrows[g, pl.ds(_LANES, _LANES)]
                out_v[g, pl.ds(64, _LANES)] = crows[g, pl.ds(0, _LANES)]
                out_v[g, pl.ds(80, _LANES)] = crows[g, pl.ds(_LANES, _LANES)]
                out_v[g, pl.ds(96, _LANES)] = a0
                out_v[g, pl.ds(112, _LANES)] = a1
                out_v[g, pl.ds(128, _LANES)] = b0
                out_v[g, pl.ds(144, _LANES)] = b1
                return 0

            lax.fori_loop(0, SUB, el_body, 0)

        pltpu.sync_copy(out_v, out_hbm.at[pl.ds(base, EPW)])

    return sc_kernel


def kernel(uid_batch_ph, mid_batch_ph, cate_batch_ph, mid_his_batch_ph,
           cate_his_batch_ph, mask, uid_table, mid_table, cate_table):
    B = uid_batch_ph.shape[0]
    L = mid_his_batch_ph.shape[1]
    n_uid, EMB = uid_table.shape
    n_mid = mid_table.shape[0]
    n_cate = cate_table.shape[0]

    sc_k = _build_sc_kernel(B, L, EMB, n_uid, n_mid, n_cate)
    return sc_k(
        uid_batch_ph,
        mid_batch_ph,
        cate_batch_ph,
        mid_his_batch_ph.reshape(B, 2, L // 2),
        cate_his_batch_ph.reshape(B, 2, L // 2),
        mask,
        uid_table,
        mid_table,
        cate_table,
    )


# speedup vs baseline: 3.1967x; 1.0939x over previous
import functools

import jax
import jax.numpy as jnp
from jax import lax
from jax.experimental import pallas as pl
from jax.experimental.pallas import tpu as pltpu
from jax.experimental.pallas import tpu_sc as plsc

_LANES = 16


def _build_sc_kernel(B, L, EMB, n_uid, n_mid, n_cate):
    info = plsc.get_sparse_core_info()
    NC, NS = info.num_cores, info.num_subcores
    NW = NC * NS
    EPW = B // NW
    SUB = 32
    NSUB = EPW // SUB
    LH = L // 2
    OUTW = 5 * EMB

    mesh = plsc.VectorSubcoreMesh(core_axis_name="c", subcore_axis_name="s")

    @functools.partial(
        pl.kernel,
        mesh=mesh,
        out_type=jax.ShapeDtypeStruct((B, OUTW), jnp.float32),
        compiler_params=pltpu.CompilerParams(use_tc_tiling_on_sc=False),
        scratch_types=[
            pltpu.VMEM((SUB, 2, LH), jnp.int32),
            pltpu.VMEM((SUB, 2, LH), jnp.int32),
            pltpu.VMEM((SUB, L), jnp.float32),
            pltpu.VMEM((L, EMB), jnp.float32),
            pltpu.VMEM((L, EMB), jnp.float32),
            pltpu.VMEM((L, EMB), jnp.float32),
            pltpu.VMEM((L, EMB), jnp.float32),
            pltpu.VMEM((EPW, OUTW), jnp.float32),
            pltpu.VMEM((EPW,), jnp.int32),
            pltpu.VMEM((EPW,), jnp.int32),
            pltpu.VMEM((EPW,), jnp.int32),
            pltpu.VMEM((EPW, EMB), jnp.float32),
            pltpu.VMEM((EPW, EMB), jnp.float32),
            pltpu.VMEM((EPW, EMB), jnp.float32),
            pltpu.SemaphoreType.DMA,
            pltpu.SemaphoreType.DMA,
            pltpu.SemaphoreType.DMA,
            pltpu.SemaphoreType.DMA,
            pltpu.SemaphoreType.DMA,
        ],
    )
    def sc_kernel(uid_i_hbm, mid_i_hbm, cate_i_hbm, mh_hbm, ch_hbm, mk_hbm,
                  uid_t_hbm, mid_t_hbm, cate_t_hbm, out_hbm,
                  mh_v, ch_v, mk_v, ra_m, ra_c, rb_m, rb_c, out_v,
                  uidx_v, midx_v, cidx_v, urows, mrows, crows,
                  sem_am, sem_ac, sem_bm, sem_bc, sem_one):
        wid = lax.axis_index("s") * NC + lax.axis_index("c")
        base = wid * EPW

        pltpu.sync_copy(uid_i_hbm.at[pl.ds(base, EPW)], uidx_v)
        pltpu.sync_copy(mid_i_hbm.at[pl.ds(base, EPW)], midx_v)
        pltpu.sync_copy(cate_i_hbm.at[pl.ds(base, EPW)], cidx_v)
        cu = pltpu.async_copy(uid_t_hbm.at[uidx_v], urows, sem_one)
        cm = pltpu.async_copy(mid_t_hbm.at[midx_v], mrows, sem_one)
        cc = pltpu.async_copy(cate_t_hbm.at[cidx_v], crows, sem_one)
        cu.wait()
        cm.wait()
        cc.wait()

        def issue(el, r_m, r_c, s_m, s_c):
            pltpu.async_copy(mid_t_hbm.at[mh_v.at[el, 0]],
                             r_m.at[pl.ds(0, LH)], s_m)
            pltpu.async_copy(mid_t_hbm.at[mh_v.at[el, 1]],
                             r_m.at[pl.ds(LH, LH)], s_m)
            pltpu.async_copy(cate_t_hbm.at[ch_v.at[el, 0]],
                             r_c.at[pl.ds(0, LH)], s_c)
            pltpu.async_copy(cate_t_hbm.at[ch_v.at[el, 1]],
                             r_c.at[pl.ds(LH, LH)], s_c)

        def drain(r_m, r_c, s_m, s_c):
            pltpu.make_async_copy(mid_t_hbm.at[pl.ds(0, L)], r_m, s_m).wait()
            pltpu.make_async_copy(cate_t_hbm.at[pl.ds(0, L)], r_c, s_c).wait()

        def reduce_store(el, g, r_m, r_c):
            zero = jnp.zeros((_LANES,), jnp.float32)

            def red_body(i, carry):
                a0, a1, b0, b1 = carry
                lb = i * _LANES
                mvec = mk_v[el, pl.ds(lb, _LANES)]
                for j in range(_LANES):
                    l = lb + j
                    m = mvec[j]
                    a0 = a0 + m * r_m[l, pl.ds(0, _LANES)]
                    a1 = a1 + m * r_m[l, pl.ds(_LANES, _LANES)]
                    b0 = b0 + m * r_c[l, pl.ds(0, _LANES)]
                    b1 = b1 + m * r_c[l, pl.ds(_LANES, _LANES)]
                return a0, a1, b0, b1

            nfull = L // _LANES
            a0, a1, b0, b1 = lax.fori_loop(
                0, nfull, red_body, (zero, zero, zero, zero))
            ntail = L - nfull * _LANES
            if ntail:
                mvec = mk_v[el, pl.ds(L - _LANES, _LANES)]
                for j in range(ntail):
                    l = nfull * _LANES + j
                    m = mvec[_LANES - ntail + j]
                    a0 = a0 + m * r_m[l, pl.ds(0, _LANES)]
                    a1 = a1 + m * r_m[l, pl.ds(_LANES, _LANES)]
                    b0 = b0 + m * r_c[l, pl.ds(0, _LANES)]
                    b1 = b1 + m * r_c[l, pl.ds(_LANES, _LANES)]

            out_v[g, pl.ds(0, _LANES)] = urows[g, pl.ds(0, _LANES)]
            out_v[g, pl.ds(16, _LANES)] = urows[g, pl.ds(_LANES, _LANES)]
            out_v[g, pl.ds(32, _LANES)] = mrows[g, pl.ds(0, _LANES)]
            out_v[g, pl.ds(48, _LANES)] = mrows[g, pl.ds(_LANES, _LANES)]
            out_v[g, pl.ds(64, _LANES)] = crows[g, pl.ds(0, _LANES)]
            out_v[g, pl.ds(80, _LANES)] = crows[g, pl.ds(_LANES, _LANES)]
            out_v[g, pl.ds(96, _LANES)] = a0
            out_v[g, pl.ds(112, _LANES)] = a1
            out_v[g, pl.ds(128, _LANES)] = b0
            out_v[g, pl.ds(144, _LANES)] = b1

        for sc in range(NSUB):
            cb = base + sc * SUB
            pltpu.sync_copy(mh_hbm.at[pl.ds(cb, SUB)], mh_v)
            pltpu.sync_copy(ch_hbm.at[pl.ds(cb, SUB)], ch_v)
            pltpu.sync_copy(mk_hbm.at[pl.ds(cb, SUB)], mk_v)

            issue(0, ra_m, ra_c, sem_am, sem_ac)

            def pair_body(p, _, sc=sc):
                e0 = 2 * p
                e1 = e0 + 1
                e2 = jnp.minimum(e0 + 2, SUB - 1)
                issue(e1, rb_m, rb_c, sem_bm, sem_bc)
                drain(ra_m, ra_c, sem_am, sem_ac)
                reduce_store(e0, sc * SUB + e0, ra_m, ra_c)
                issue(e2, ra_m, ra_c, sem_am, sem_ac)
                drain(rb_m, rb_c, sem_bm, sem_bc)
                reduce_store(e1, sc * SUB + e1, rb_m, rb_c)
                return 0

            lax.fori_loop(0, SUB // 2, pair_body, 0)
            drain(ra_m, ra_c, sem_am, sem_ac)

        pltpu.sync_copy(out_v, out_hbm.at[pl.ds(base, EPW)])

    return sc_kernel


def kernel(uid_batch_ph, mid_batch_ph, cate_batch_ph, mid_his_batch_ph,
           cate_his_batch_ph, mask, uid_table, mid_table, cate_table):
    B = uid_batch_ph.shape[0]
    L = mid_his_batch_ph.shape[1]
    n_uid, EMB = uid_table.shape
    n_mid = mid_table.shape[0]
    n_cate = cate_table.shape[0]

    sc_k = _build_sc_kernel(B, L, EMB, n_uid, n_mid, n_cate)
    return sc_k(
        uid_batch_ph,
        mid_batch_ph,
        cate_batch_ph,
        mid_his_batch_ph.reshape(B, 2, L // 2),
        cate_his_batch_ph.reshape(B, 2, L // 2),
        mask,
        uid_table,
        mid_table,
        cate_table,
    )
